# Initial kernel scaffold; baseline (speedup 1.0000x reference)
import functools

import jax
import jax.numpy as jnp
from jax import lax
from jax.experimental import pallas as pl
from jax.experimental.pallas import tpu as pltpu
from jax.experimental.pallas import tpu_sc as plsc

N_NODES = 10000
N_EDGES = 160000
D = 128
NUM_RELS = 8
RRELU_SLOPE = (1.0 / 8.0 + 1.0 / 3.0) / 2.0

NC, NS = 2, 16
NW = NC * NS
CHUNK = 128
NCHUNK = N_EDGES // CHUNK
ROWS_PER_SUB = N_NODES // NS

B_EDGE = 2000
NBLK = N_EDGES // B_EDGE

_SC_MESH = plsc.VectorSubcoreMesh(
    core_axis_name="c", subcore_axis_name="s", num_cores=NC, num_subcores=NS)


def _zero_rows(ref, nrows, ncols):
  @pl.loop(0, nrows)
  def _(i):
    for c in range(ncols // 16):
      ref[i, pl.ds(c * 16, 16)] = jnp.zeros((16,), jnp.float32)


def _hist_kernel(src, dst, et):
  @functools.partial(
      pl.kernel,
      out_type=jax.ShapeDtypeStruct((NC, N_NODES, 16), jnp.float32),
      mesh=_SC_MESH,
      scratch_types=[
          pltpu.VMEM((CHUNK,), jnp.int32),
          pltpu.VMEM((CHUNK,), jnp.int32),
          pltpu.VMEM((CHUNK,), jnp.int32),
          pltpu.VMEM((CHUNK, 16), jnp.float32),
          pltpu.VMEM((CHUNK, 16), jnp.float32),
          pltpu.VMEM_SHARED((N_NODES, 16), jnp.float32),
          pltpu.SemaphoreType.DMA,
      ],
  )
  def k(src_hbm, dst_hbm, et_hbm, h_hbm, srcv, dstv, etv, ohs, ohd, hsh, sem):
    cid = lax.axis_index("c")
    sid = lax.axis_index("s")
    wid = sid * NC + cid

    _zero_rows(ohs, CHUNK, 16)
    @pl.loop(0, 5)
    def _(i):
      pltpu.sync_copy(ohs.at[pl.ds(0, 125)],
                      hsh.at[pl.ds(sid * ROWS_PER_SUB + i * 125, 125)])
    plsc.subcore_barrier()

    iota16 = lax.iota(jnp.int32, 16)
    ones16 = jnp.ones((16,), jnp.float32)

    @pl.loop(wid, NCHUNK, step=NW)
    def _(kc):
      base = kc * CHUNK
      pltpu.sync_copy(src_hbm.at[pl.ds(base, CHUNK)], srcv)
      pltpu.sync_copy(dst_hbm.at[pl.ds(base, CHUNK)], dstv)
      pltpu.sync_copy(et_hbm.at[pl.ds(base, CHUNK)], etv)
      _zero_rows(ohs, CHUNK, 16)
      _zero_rows(ohd, CHUNK, 16)
      @pl.loop(0, CHUNK, step=16)
      def _(j):
        rows = iota16 + j
        e16 = etv[pl.ds(j, 16)]
        plsc.store_scatter(ohs, [rows, e16], ones16)
        plsc.store_scatter(ohd, [rows, e16], ones16)
        plsc.store_scatter(ohd, [rows, e16 + 8], ones16)
      pltpu.sync_copy(ohs, hsh.at[srcv], add=True)
      pltpu.sync_copy(ohd, hsh.at[dstv], add=True)

    plsc.subcore_barrier()
    pltpu.sync_copy(hsh.at[pl.ds(sid * ROWS_PER_SUB, ROWS_PER_SUB)],
                    h_hbm.at[cid, pl.ds(sid * ROWS_PER_SUB, ROWS_PER_SUB)])

  return k(src, dst, et)


def _gather_kernel(nf, ef, src, dst):
  @functools.partial(
      pl.kernel,
      out_type=(
          jax.ShapeDtypeStruct((N_EDGES, D), jnp.float32),
          jax.ShapeDtypeStruct((N_EDGES, D), jnp.float32),
          jax.ShapeDtypeStruct((N_NODES, D), jnp.float32),
          jax.ShapeDtypeStruct((N_NODES, D), jnp.float32),
      ),
      mesh=_SC_MESH,
      scratch_types=[
          pltpu.VMEM((CHUNK,), jnp.int32),
          pltpu.VMEM((CHUNK,), jnp.int32),
          pltpu.VMEM((CHUNK, D), jnp.float32),
          pltpu.VMEM((CHUNK, D), jnp.float32),
          pltpu.VMEM_SHARED((N_NODES, D), jnp.float32),
          pltpu.SemaphoreType.DMA,
      ],
  )
  def k(nf_hbm, ef_hbm, src_hbm, dst_hbm, g0_hbm, g1_hbm, s1_hbm, s2_hbm,
        idxa, idxb, rows, erows, acc, sem):
    cid = lax.axis_index("c")
    sid = lax.axis_index("s")

    _zero_rows(rows, CHUNK, D)
    @pl.loop(0, 5)
    def _(i):
      pltpu.sync_copy(rows.at[pl.ds(0, 125)],
                      acc.at[pl.ds(sid * ROWS_PER_SUB + i * 125, 125)])
    plsc.subcore_barrier()

    @pl.when(cid == 0)
    def _():
      @pl.loop(sid, NCHUNK, step=NS)
      def _(kc):
        base = kc * CHUNK
        pltpu.sync_copy(src_hbm.at[pl.ds(base, CHUNK)], idxa)
        pltpu.sync_copy(dst_hbm.at[pl.ds(base, CHUNK)], idxb)
        pltpu.async_copy(nf_hbm.at[idxa], rows, sem).wait()
        pltpu.sync_copy(rows, g0_hbm.at[pl.ds(base, CHUNK)])
        pltpu.sync_copy(rows, acc.at[idxb], add=True)
      plsc.subcore_barrier()
      pltpu.sync_copy(acc.at[pl.ds(sid * ROWS_PER_SUB, ROWS_PER_SUB)],
                      s1_hbm.at[pl.ds(sid * ROWS_PER_SUB, ROWS_PER_SUB)])

    @pl.when(cid == 1)
    def _():
      @pl.loop(sid, NCHUNK, step=NS)
      def _(kc):
        base = kc * CHUNK
        pltpu.sync_copy(dst_hbm.at[pl.ds(base, CHUNK)], idxb)
        pltpu.async_copy(nf_hbm.at[idxb], rows, sem).wait()
        pltpu.sync_copy(rows, g1_hbm.at[pl.ds(base, CHUNK)])
        pltpu.sync_copy(ef_hbm.at[pl.ds(base, CHUNK)], erows)
        pltpu.sync_copy(erows, acc.at[idxb], add=True)
      plsc.subcore_barrier()
      pltpu.sync_copy(acc.at[pl.ds(sid * ROWS_PER_SUB, ROWS_PER_SUB)],
                      s2_hbm.at[pl.ds(sid * ROWS_PER_SUB, ROWS_PER_SUB)])

  return k(nf, ef, src, dst)


def _prep_body(h_ref, nf_ref, w1aT_ref, b1_ref, w2aT_ref, b2_ref,
               re1_ref, re2_ref):
  f32 = jnp.float32
  hs = h_ref[0] + h_ref[1]
  maskT = (hs[:, 0:NUM_RELS] > 0).astype(f32)
  counts = jnp.sum(maskT, axis=0)
  nf = nf_ref[...]
  sums = []
  for r in range(NUM_RELS):
    sums.append(jnp.sum(nf * maskT[:, r:r + 1], axis=0, keepdims=True))
  sums = jnp.concatenate(sums, axis=0)
  rel = jnp.where(counts[:, None] > 0,
                  sums / jnp.maximum(counts, 1.0)[:, None], 0.0)
  re1_ref[...] = jnp.dot(rel, w1aT_ref[...],
                         preferred_element_type=f32) + b1_ref[...]
  re2_ref[...] = jnp.dot(rel, w2aT_ref[...],
                         preferred_element_type=f32) + b2_ref[...]


def _prep(h2, nf, w1aT, b1, w2aT, b2):
  return pl.pallas_call(
      _prep_body,
      out_shape=(jax.ShapeDtypeStruct((NUM_RELS, D), jnp.float32),
                 jax.ShapeDtypeStruct((NUM_RELS, D), jnp.float32)),
  )(h2, nf, w1aT, b1, w2aT, b2)


def _gru_combine(gi, gh, h):
  ir, iz, inn = gi[:, 0:D], gi[:, D:2 * D], gi[:, 2 * D:3 * D]
  hr, hz, hn = gh[:, 0:D], gh[:, D:2 * D], gh[:, 2 * D:3 * D]
  r = jax.nn.sigmoid(ir + hr)
  z = jax.nn.sigmoid(iz + hz)
  n = jnp.tanh(inn + r * hn)
  return (1.0 - z) * n + z * h


def _edge_body(et_ref, g0_ref, g1_ref, ef_ref, re2_ref, w2bT_ref, w2cT_ref,
               wihT_ref, whhT_ref, bih_ref, bhh_ref, out_ref):
  f32, bf16 = jnp.float32, jnp.bfloat16
  ef = ef_ref[...]
  efb = ef.astype(bf16)
  oh = (et_ref[...] == lax.broadcasted_iota(
      jnp.int32, (B_EDGE, NUM_RELS), 1)).astype(f32)
  x2 = jnp.dot(oh, re2_ref[...], preferred_element_type=f32)
  x2 = x2 + jnp.dot(g0_ref[...].astype(bf16), w2bT_ref[...].astype(bf16),
                    preferred_element_type=f32)
  x2 = x2 + jnp.dot(g1_ref[...].astype(bf16), w2cT_ref[...].astype(bf16),
                    preferred_element_type=f32)
  wihT = wihT_ref[...].astype(bf16)
  bih = bih_ref[...]
  gh = jnp.dot(efb, whhT_ref[...].astype(bf16),
               preferred_element_type=f32) + bhh_ref[...]
  gi = jnp.dot(x2.astype(bf16), wihT, preferred_element_type=f32) + bih
  e_msg = _gru_combine(gi, gh, ef)
  gi2 = jnp.dot(e_msg.astype(bf16), wihT, preferred_element_type=f32) + bih
  out_ref[...] = _gru_combine(gi2, gh, ef)


def _edge(et_col, g0, g1, ef, re2, w2bT, w2cT, wihT, whhT, bih, bhh):
  full = lambda shape: pl.BlockSpec(shape, lambda i: (0, 0))
  blk = lambda shape: pl.BlockSpec(shape, lambda i: (i, 0))
  return pl.pallas_call(
      _edge_body,
      grid=(NBLK,),
      in_specs=[
          blk((B_EDGE, 1)),
          blk((B_EDGE, D)),
          blk((B_EDGE, D)),
          blk((B_EDGE, D)),
          full((NUM_RELS, D)),
          full((D, D)),
          full((D, D)),
          full((D, 3 * D)),
          full((D, 3 * D)),
          full((1, 3 * D)),
          full((1, 3 * D)),
      ],
      out_specs=blk((B_EDGE, D)),
      out_shape=jax.ShapeDtypeStruct((N_EDGES, D), jnp.float32),
  )(et_col, g0, g1, ef, re2, w2bT, w2cT, wihT, whhT, bih, bhh)


def _node_body(nf_ref, s1_ref, s2_ref, h_ref, re1_ref, w1bT_ref, w1cT_ref,
               w3T_ref, b3_ref, out_ref):
  f32 = jnp.float32
  hs = h_ref[0] + h_ref[1]
  cnt2 = hs[:, NUM_RELS:2 * NUM_RELS]
  deg = jnp.sum(cnt2, axis=1, keepdims=True)
  aggs = jnp.dot(cnt2, re1_ref[...], preferred_element_type=f32)
  aggs = aggs + jnp.dot(s1_ref[...], w1bT_ref[...], preferred_element_type=f32)
  aggs = aggs + jnp.dot(s2_ref[...], w1cT_ref[...], preferred_element_type=f32)
  agg = aggs / jnp.maximum(deg, 1.0)
  x = agg + jnp.dot(nf_ref[...], w3T_ref[...],
                    preferred_element_type=f32) + b3_ref[...]
  out_ref[...] = jnp.where(x >= 0, x, RRELU_SLOPE * x)


def _node(nf, s1, s2, h2, re1, w1bT, w1cT, w3T, b3):
  return pl.pallas_call(
      _node_body,
      out_shape=jax.ShapeDtypeStruct((N_NODES, D), jnp.float32),
  )(nf, s1, s2, h2, re1, w1bT, w1cT, w3T, b3)


def kernel(node_feats, edge_index, edge_feats, edge_types, params):
  src = edge_index[0]
  dst = edge_index[1]
  et = edge_types
  et_col = et.reshape(N_EDGES, 1)

  h2 = _hist_kernel(src, dst, et)

  nf, ef = node_feats, edge_feats
  for p in params:
    w1, w2 = p['W1'], p['W2']
    w1aT, w1bT, w1cT = w1[:, 0:D].T, w1[:, D:2 * D].T, w1[:, 2 * D:3 * D].T
    w2aT, w2bT, w2cT = w2[:, 0:D].T, w2[:, D:2 * D].T, w2[:, 2 * D:3 * D].T
    b1 = p['b1'].reshape(1, D)
    b2 = p['b2'].reshape(1, D)
    b3 = p['b3'].reshape(1, D)
    w3T = p['W3'].T
    wihT = p['Wih'].T
    whhT = p['Whh'].T
    bih = p['bih'].reshape(1, 3 * D)
    bhh = p['bhh'].reshape(1, 3 * D)

    g0, g1, s1, s2 = _gather_kernel(nf, ef, src, dst)
    re1, re2 = _prep(h2, nf, w1aT, b1, w2aT, b2)
    new_e = _edge(et_col, g0, g1, ef, re2, w2bT, w2cT, wihT, whhT, bih, bhh)
    new_n = _node(nf, s1, s2, h2, re1, w1bT, w1cT, w3T, b3)
    nf, ef = new_n, new_e
  return (nf, ef)


# speedup vs baseline: 2.4272x; 2.4272x over previous
import functools

import jax
import jax.numpy as jnp
from jax import lax
from jax.experimental import pallas as pl
from jax.experimental.pallas import tpu as pltpu
from jax.experimental.pallas import tpu_sc as plsc

N_NODES = 10000
N_EDGES = 160000
D = 128
NUM_RELS = 8
RRELU_SLOPE = (1.0 / 8.0 + 1.0 / 3.0) / 2.0

NC, NS = 2, 16
NW = NC * NS
CHUNK = 128
NCHUNK = N_EDGES // CHUNK
ROWS_PER_SUB = N_NODES // NS

B_EDGE = 2000
NBLK = N_EDGES // B_EDGE

def _sc_mesh():
  return plsc.VectorSubcoreMesh(
      core_axis_name="c", subcore_axis_name="s", num_cores=NC, num_subcores=NS)


def _sc_params():
  import dataclasses
  cp = pltpu.CompilerParams()
  if "needs_layout_passes" in pltpu.CompilerParams.__dataclass_fields__:
    cp = dataclasses.replace(cp, needs_layout_passes=False)
  return cp


def _zero_rows(ref, nrows, ncols):
  @pl.loop(0, nrows)
  def _(i):
    for c in range(ncols // 16):
      ref[i, pl.ds(c * 16, 16)] = jnp.zeros((16,), jnp.float32)


_SUB_ROWS = 624
_TAIL_START = _SUB_ROWS * NS
_TAIL_ROWS = N_NODES - _TAIL_START


def _zero_shared_range(zbuf, shared, sid):
  start = pl.multiple_of(sid * _SUB_ROWS, 8)
  for i in range(4):
    pltpu.sync_copy(zbuf, shared.at[pl.ds(start + i * 128, 128)])
  pltpu.sync_copy(zbuf.at[pl.ds(0, 112)], shared.at[pl.ds(start + 512, 112)])
  @pl.when(sid == NS - 1)
  def _():
    pltpu.sync_copy(zbuf.at[pl.ds(0, _TAIL_ROWS)],
                    shared.at[pl.ds(_TAIL_START, _TAIL_ROWS)])


def _writeout_shared(shared, out_ref, sid):
  start = pl.multiple_of(sid * _SUB_ROWS, 8)
  pltpu.sync_copy(shared.at[pl.ds(start, _SUB_ROWS)],
                  out_ref.at[pl.ds(start, _SUB_ROWS)])
  @pl.when(sid == NS - 1)
  def _():
    pltpu.sync_copy(shared.at[pl.ds(_TAIL_START, _TAIL_ROWS)],
                    out_ref.at[pl.ds(_TAIL_START, _TAIL_ROWS)])


def _hist_kernel(src, dst, et, t1, t2):
  @functools.partial(
      pl.kernel,
      out_type=jax.ShapeDtypeStruct((NC, N_NODES, D), jnp.float32),
      mesh=_sc_mesh(),
      compiler_params=_sc_params(),
      scratch_types=[
          pltpu.VMEM((CHUNK,), jnp.int32),
          pltpu.VMEM((CHUNK,), jnp.int32),
          pltpu.VMEM((CHUNK, D), jnp.float32),
          pltpu.VMEM_SHARED((N_NODES, D), jnp.float32),
          pltpu.SemaphoreType.DMA,
      ],
  )
  def k(t1_hbm, t2_hbm, src_hbm, dst_hbm, et_hbm, h_hbm,
        idxa, idxb, rows, acc, sem):
    cid = lax.axis_index("c")
    sid = lax.axis_index("s")

    _zero_rows(rows, CHUNK, D)
    _zero_shared_range(rows, acc, sid)
    plsc.subcore_barrier()

    @pl.when(cid == 0)
    def _():
      @pl.loop(sid, NCHUNK, step=NS)
      def _(kc):
        base = kc * CHUNK
        pltpu.sync_copy(et_hbm.at[pl.ds(base, CHUNK)], idxa)
        pltpu.sync_copy(src_hbm.at[pl.ds(base, CHUNK)], idxb)
        pltpu.async_copy(t1_hbm.at[idxa], rows, sem).wait()
        pltpu.sync_copy(rows, acc.at[idxb], add=True)
      plsc.subcore_barrier()
      _writeout_shared(acc, h_hbm.at[0], sid)

    @pl.when(cid == 1)
    def _():
      @pl.loop(sid, NCHUNK, step=NS)
      def _(kc):
        base = kc * CHUNK
        pltpu.sync_copy(et_hbm.at[pl.ds(base, CHUNK)], idxa)
        pltpu.sync_copy(dst_hbm.at[pl.ds(base, CHUNK)], idxb)
        pltpu.async_copy(t2_hbm.at[idxa], rows, sem).wait()
        pltpu.sync_copy(rows, acc.at[idxb], add=True)
      plsc.subcore_barrier()
      _writeout_shared(acc, h_hbm.at[1], sid)

  return k(t1, t2, src, dst, et)


def _gather_kernel(nf, ef, src, dst):
  @functools.partial(
      pl.kernel,
      out_type=(
          jax.ShapeDtypeStruct((N_EDGES, D), jnp.float32),
          jax.ShapeDtypeStruct((N_EDGES, D), jnp.float32),
          jax.ShapeDtypeStruct((N_NODES, D), jnp.float32),
          jax.ShapeDtypeStruct((N_NODES, D), jnp.float32),
      ),
      mesh=_sc_mesh(),
      compiler_params=_sc_params(),
      scratch_types=[
          pltpu.VMEM((CHUNK,), jnp.int32),
          pltpu.VMEM((CHUNK,), jnp.int32),
          pltpu.VMEM((CHUNK, D), jnp.float32),
          pltpu.VMEM((CHUNK, D), jnp.float32),
          pltpu.VMEM_SHARED((N_NODES, D), jnp.float32),
          pltpu.SemaphoreType.DMA,
      ],
  )
  def k(nf_hbm, ef_hbm, src_hbm, dst_hbm, g0_hbm, g1_hbm, s1_hbm, s2_hbm,
        idxa, idxb, rows, erows, acc, sem):
    cid = lax.axis_index("c")
    sid = lax.axis_index("s")

    _zero_rows(rows, CHUNK, D)
    _zero_shared_range(rows, acc, sid)
    plsc.subcore_barrier()

    @pl.when(cid == 0)
    def _():
      @pl.loop(sid, NCHUNK, step=NS)
      def _(kc):
        base = kc * CHUNK
        pltpu.sync_copy(src_hbm.at[pl.ds(base, CHUNK)], idxa)
        pltpu.sync_copy(dst_hbm.at[pl.ds(base, CHUNK)], idxb)
        pltpu.async_copy(nf_hbm.at[idxa], rows, sem).wait()
        pltpu.sync_copy(rows, g0_hbm.at[pl.ds(base, CHUNK)])
        pltpu.sync_copy(rows, acc.at[idxb], add=True)
      plsc.subcore_barrier()
      _writeout_shared(acc, s1_hbm, sid)

    @pl.when(cid == 1)
    def _():
      @pl.loop(sid, NCHUNK, step=NS)
      def _(kc):
        base = kc * CHUNK
        pltpu.sync_copy(dst_hbm.at[pl.ds(base, CHUNK)], idxb)
        pltpu.async_copy(nf_hbm.at[idxb], rows, sem).wait()
        pltpu.sync_copy(rows, g1_hbm.at[pl.ds(base, CHUNK)])
        pltpu.sync_copy(ef_hbm.at[pl.ds(base, CHUNK)], erows)
        pltpu.sync_copy(erows, acc.at[idxb], add=True)
      plsc.subcore_barrier()
      _writeout_shared(acc, s2_hbm, sid)

  return k(nf, ef, src, dst)


def _prep_body(h_ref, nf_ref, w1aT_ref, b1_ref, w2aT_ref, b2_ref,
               re1_ref, re2_ref):
  f32 = jnp.float32
  hs = h_ref[0] + h_ref[1]
  maskT = (hs[:, 0:NUM_RELS] > 0).astype(f32)
  counts = jnp.sum(maskT, axis=0)
  nf = nf_ref[...]
  sums = []
  for r in range(NUM_RELS):
    sums.append(jnp.sum(nf * maskT[:, r:r + 1], axis=0, keepdims=True))
  sums = jnp.concatenate(sums, axis=0)
  rel = jnp.where(counts[:, None] > 0,
                  sums / jnp.maximum(counts, 1.0)[:, None], 0.0)
  re1_ref[...] = jnp.dot(rel, w1aT_ref[...],
                         preferred_element_type=f32) + b1_ref[...]
  re2_ref[...] = jnp.dot(rel, w2aT_ref[...],
                         preferred_element_type=f32) + b2_ref[...]


def _prep(h2, nf, w1aT, b1, w2aT, b2):
  return pl.pallas_call(
      _prep_body,
      out_shape=(jax.ShapeDtypeStruct((NUM_RELS, D), jnp.float32),
                 jax.ShapeDtypeStruct((NUM_RELS, D), jnp.float32)),
  )(h2, nf, w1aT, b1, w2aT, b2)


def _gru_combine(gi, gh, h):
  ir, iz, inn = gi[:, 0:D], gi[:, D:2 * D], gi[:, 2 * D:3 * D]
  hr, hz, hn = gh[:, 0:D], gh[:, D:2 * D], gh[:, 2 * D:3 * D]
  r = jax.nn.sigmoid(ir + hr)
  z = jax.nn.sigmoid(iz + hz)
  n = jnp.tanh(inn + r * hn)
  return (1.0 - z) * n + z * h


def _edge_body(et_ref, g0_ref, g1_ref, ef_ref, re2_ref, w2bT_ref, w2cT_ref,
               wihT_ref, whhT_ref, bih_ref, bhh_ref, out_ref):
  f32, bf16 = jnp.float32, jnp.bfloat16
  ef = ef_ref[...]
  efb = ef.astype(bf16)
  oh = (et_ref[...] == lax.broadcasted_iota(
      jnp.int32, (B_EDGE, NUM_RELS), 1)).astype(f32)
  x2 = jnp.dot(oh, re2_ref[...], preferred_element_type=f32)
  x2 = x2 + jnp.dot(g0_ref[...].astype(bf16), w2bT_ref[...].astype(bf16),
                    preferred_element_type=f32)
  x2 = x2 + jnp.dot(g1_ref[...].astype(bf16), w2cT_ref[...].astype(bf16),
                    preferred_element_type=f32)
  wihT = wihT_ref[...].astype(bf16)
  bih = bih_ref[...]
  gh = jnp.dot(efb, whhT_ref[...].astype(bf16),
               preferred_element_type=f32) + bhh_ref[...]
  gi = jnp.dot(x2.astype(bf16), wihT, preferred_element_type=f32) + bih
  e_msg = _gru_combine(gi, gh, ef)
  gi2 = jnp.dot(e_msg.astype(bf16), wihT, preferred_element_type=f32) + bih
  out_ref[...] = _gru_combine(gi2, gh, ef)


def _edge(et_col, g0, g1, ef, re2, w2bT, w2cT, wihT, whhT, bih, bhh):
  full = lambda shape: pl.BlockSpec(shape, lambda i: (0, 0))
  blk = lambda shape: pl.BlockSpec(shape, lambda i: (i, 0))
  return pl.pallas_call(
      _edge_body,
      grid=(NBLK,),
      in_specs=[
          blk((B_EDGE, 1)),
          blk((B_EDGE, D)),
          blk((B_EDGE, D)),
          blk((B_EDGE, D)),
          full((NUM_RELS, D)),
          full((D, D)),
          full((D, D)),
          full((D, 3 * D)),
          full((D, 3 * D)),
          full((1, 3 * D)),
          full((1, 3 * D)),
      ],
      out_specs=blk((B_EDGE, D)),
      out_shape=jax.ShapeDtypeStruct((N_EDGES, D), jnp.float32),
  )(et_col, g0, g1, ef, re2, w2bT, w2cT, wihT, whhT, bih, bhh)


def _node_body(nf_ref, s1_ref, s2_ref, h_ref, re1_ref, w1bT_ref, w1cT_ref,
               w3T_ref, b3_ref, out_ref):
  f32 = jnp.float32
  hs = h_ref[0] + h_ref[1]
  cnt2 = hs[:, NUM_RELS:2 * NUM_RELS]
  deg = jnp.sum(cnt2, axis=1, keepdims=True)
  aggs = jnp.dot(cnt2, re1_ref[...], preferred_element_type=f32)
  aggs = aggs + jnp.dot(s1_ref[...], w1bT_ref[...], preferred_element_type=f32)
  aggs = aggs + jnp.dot(s2_ref[...], w1cT_ref[...], preferred_element_type=f32)
  agg = aggs / jnp.maximum(deg, 1.0)
  x = agg + jnp.dot(nf_ref[...], w3T_ref[...],
                    preferred_element_type=f32) + b3_ref[...]
  out_ref[...] = jnp.where(x >= 0, x, RRELU_SLOPE * x)


def _node(nf, s1, s2, h2, re1, w1bT, w1cT, w3T, b3):
  return pl.pallas_call(
      _node_body,
      out_shape=jax.ShapeDtypeStruct((N_NODES, D), jnp.float32),
  )(nf, s1, s2, h2, re1, w1bT, w1cT, w3T, b3)


def kernel(node_feats, edge_index, edge_feats, edge_types, params):
  src = edge_index[0]
  dst = edge_index[1]
  et = edge_types
  et_col = et.reshape(N_EDGES, 1)

  eye8 = jnp.eye(NUM_RELS, dtype=jnp.float32)
  t1 = jnp.pad(eye8, ((0, 0), (0, D - NUM_RELS)))
  t2 = t1 + jnp.pad(eye8, ((0, 0), (NUM_RELS, D - 2 * NUM_RELS)))
  h2 = _hist_kernel(src, dst, et, t1, t2)

  nf, ef = node_feats, edge_feats
  for p in params:
    w1, w2 = p['W1'], p['W2']
    w1aT, w1bT, w1cT = w1[:, 0:D].T, w1[:, D:2 * D].T, w1[:, 2 * D:3 * D].T
    w2aT, w2bT, w2cT = w2[:, 0:D].T, w2[:, D:2 * D].T, w2[:, 2 * D:3 * D].T
    b1 = p['b1'].reshape(1, D)
    b2 = p['b2'].reshape(1, D)
    b3 = p['b3'].reshape(1, D)
    w3T = p['W3'].T
    wihT = p['Wih'].T
    whhT = p['Whh'].T
    bih = p['bih'].reshape(1, 3 * D)
    bhh = p['bhh'].reshape(1, 3 * D)

    g0, g1, s1, s2 = _gather_kernel(nf, ef, src, dst)
    re1, re2 = _prep(h2, nf, w1aT, b1, w2aT, b2)
    new_e = _edge(et_col, g0, g1, ef, re2, w2bT, w2cT, wihT, whhT, bih, bhh)
    new_n = _node(nf, s1, s2, h2, re1, w1bT, w1cT, w3T, b3)
    nf, ef = new_n, new_e
  return (nf, ef)


# speedup vs baseline: 2.8914x; 1.1913x over previous
import functools

import jax
import jax.numpy as jnp
from jax import lax
from jax.experimental import pallas as pl
from jax.experimental.pallas import tpu as pltpu
from jax.experimental.pallas import tpu_sc as plsc

N_NODES = 10000
N_EDGES = 160000
D = 128
NUM_RELS = 8
RRELU_SLOPE = (1.0 / 8.0 + 1.0 / 3.0) / 2.0

NC, NS = 2, 16
NW = NC * NS
CH = 80
EDGES_PER_SUB = N_EDGES // NS
NCH = EDGES_PER_SUB // CH
ROWS_PER_SUB = N_NODES // NS

B_EDGE = 2000
NBLK = N_EDGES // B_EDGE

def _sc_mesh():
  return plsc.VectorSubcoreMesh(
      core_axis_name="c", subcore_axis_name="s", num_cores=NC, num_subcores=NS)


def _sc_params():
  import dataclasses
  cp = pltpu.CompilerParams()
  if "needs_layout_passes" in pltpu.CompilerParams.__dataclass_fields__:
    cp = dataclasses.replace(cp, needs_layout_passes=False)
  return cp


def _zero_rows(ref, nrows, ncols):
  @pl.loop(0, nrows)
  def _(i):
    for c in range(ncols // 16):
      ref[i, pl.ds(c * 16, 16)] = jnp.zeros((16,), jnp.float32)


def _pipeline(nch, idx_fire, idx_wait, s2_fire, s2_wait, work_fire, work_wait):
  assert nch >= 4 and nch % 2 == 1
  idx_fire(0, 0)
  idx_fire(1, 1)
  idx_wait(0)
  s2_fire(0, 0)

  def body(k, p, mid, tail):
    q = 1 - p
    s2_wait(p)
    work_fire(k, p)
    if mid:
      idx_wait(q)
      s2_fire(k + 1, q)
    work_wait(p)
    if tail:
      idx_fire(k + 2, p)

  @pl.loop(0, (nch - 3) // 2)
  def _(t):
    body(2 * t, 0, True, True)
    body(2 * t + 1, 1, True, True)

  body(nch - 3, 0, True, True)
  body(nch - 2, 1, True, False)
  body(nch - 1, 0, False, False)


_SUB_ROWS = 624
_TAIL_START = _SUB_ROWS * NS
_TAIL_ROWS = N_NODES - _TAIL_START


def _zero_shared_range(zbuf, shared, sid):
  start = pl.multiple_of(sid * _SUB_ROWS, 8)
  for i in range(7):
    pltpu.sync_copy(zbuf, shared.at[pl.ds(start + i * CH, CH)])
  pltpu.sync_copy(zbuf.at[pl.ds(0, 64)], shared.at[pl.ds(start + 560, 64)])
  @pl.when(sid == NS - 1)
  def _():
    pltpu.sync_copy(zbuf.at[pl.ds(0, _TAIL_ROWS)],
                    shared.at[pl.ds(_TAIL_START, _TAIL_ROWS)])


def _writeout_shared(shared, out_ref, sid):
  start = pl.multiple_of(sid * _SUB_ROWS, 8)
  pltpu.sync_copy(shared.at[pl.ds(start, _SUB_ROWS)],
                  out_ref.at[pl.ds(start, _SUB_ROWS)])
  @pl.when(sid == NS - 1)
  def _():
    pltpu.sync_copy(shared.at[pl.ds(_TAIL_START, _TAIL_ROWS)],
                    out_ref.at[pl.ds(_TAIL_START, _TAIL_ROWS)])


def _hist_kernel(src, dst, et, t1, t2):
  @functools.partial(
      pl.kernel,
      out_type=jax.ShapeDtypeStruct((NC, N_NODES, D), jnp.float32),
      mesh=_sc_mesh(),
      compiler_params=_sc_params(),
      scratch_types=[
          [pltpu.VMEM((CH,), jnp.int32)] * 2,
          [pltpu.VMEM((CH,), jnp.int32)] * 2,
          [pltpu.VMEM((CH, D), jnp.float32)] * 2,
          pltpu.VMEM_SHARED((N_NODES, D), jnp.float32),
          [pltpu.SemaphoreType.DMA] * 2,
          [pltpu.SemaphoreType.DMA] * 2,
          [pltpu.SemaphoreType.DMA] * 2,
      ],
  )
  def k(t1_hbm, t2_hbm, src_hbm, dst_hbm, et_hbm, h_hbm,
        iet, ind, rows, acc, semi, semg, semw):
    cid = lax.axis_index("c")
    sid = lax.axis_index("s")
    estart = sid * EDGES_PER_SUB

    _zero_rows(rows[0], CH, D)
    _zero_shared_range(rows[0], acc, sid)
    plsc.subcore_barrier()

    def job(t_hbm, nd_hbm, out_ref):
      def idx_fire(kc, p):
        base = estart + kc * CH
        pltpu.async_copy(et_hbm.at[pl.ds(base, CH)], iet[p], semi[p])
        pltpu.async_copy(nd_hbm.at[pl.ds(base, CH)], ind[p], semi[p])
      def idx_wait(p):
        pltpu.make_async_copy(et_hbm.at[pl.ds(0, CH)], iet[p], semi[p]).wait()
        pltpu.make_async_copy(nd_hbm.at[pl.ds(0, CH)], ind[p], semi[p]).wait()
      def s2_fire(kc, p):
        pltpu.async_copy(t_hbm.at[iet[p]], rows[p], semg[p])
      def s2_wait(p):
        pltpu.make_async_copy(t_hbm.at[iet[p]], rows[p], semg[p]).wait()
      def work_fire(kc, p):
        pltpu.async_copy(rows[p], acc.at[ind[p]], semw[p], add=True)
      def work_wait(p):
        pltpu.make_async_copy(rows[p], acc.at[ind[p]], semw[p]).wait()
      _pipeline(NCH, idx_fire, idx_wait, s2_fire, s2_wait, work_fire, work_wait)
      plsc.subcore_barrier()
      _writeout_shared(acc, out_ref, sid)

    @pl.when(cid == 0)
    def _():
      job(t1_hbm, src_hbm, h_hbm.at[0])

    @pl.when(cid == 1)
    def _():
      job(t2_hbm, dst_hbm, h_hbm.at[1])

  return k(t1, t2, src, dst, et)


def _gather_kernel(nf, ef, src, dst):
  @functools.partial(
      pl.kernel,
      out_type=(
          jax.ShapeDtypeStruct((N_EDGES, D), jnp.float32),
          jax.ShapeDtypeStruct((N_EDGES, D), jnp.float32),
          jax.ShapeDtypeStruct((N_NODES, D), jnp.float32),
          jax.ShapeDtypeStruct((N_NODES, D), jnp.float32),
      ),
      mesh=_sc_mesh(),
      compiler_params=_sc_params(),
      scratch_types=[
          [pltpu.VMEM((CH,), jnp.int32)] * 2,
          [pltpu.VMEM((CH,), jnp.int32)] * 2,
          [pltpu.VMEM((CH, D), jnp.float32)] * 2,
          [pltpu.VMEM((CH, D), jnp.float32)] * 2,
          pltpu.VMEM_SHARED((N_NODES, D), jnp.float32),
          [pltpu.SemaphoreType.DMA] * 2,
          [pltpu.SemaphoreType.DMA] * 2,
          [pltpu.SemaphoreType.DMA] * 2,
          [pltpu.SemaphoreType.DMA] * 2,
          [pltpu.SemaphoreType.DMA] * 2,
      ],
  )
  def k(nf_hbm, ef_hbm, src_hbm, dst_hbm, g0_hbm, g1_hbm, s1_hbm, s2_hbm,
        isrc, idst, rows, erows, acc, semi, semg, seme, semw, sema):
    cid = lax.axis_index("c")
    sid = lax.axis_index("s")
    estart = sid * EDGES_PER_SUB

    _zero_rows(rows[0], CH, D)
    _zero_shared_range(rows[0], acc, sid)
    plsc.subcore_barrier()

    @pl.when(cid == 0)
    def _():
      def idx_fire(kc, p):
        base = estart + kc * CH
        pltpu.async_copy(src_hbm.at[pl.ds(base, CH)], isrc[p], semi[p])
        pltpu.async_copy(dst_hbm.at[pl.ds(base, CH)], idst[p], semi[p])
      def idx_wait(p):
        pltpu.make_async_copy(src_hbm.at[pl.ds(0, CH)], isrc[p], semi[p]).wait()
        pltpu.make_async_copy(dst_hbm.at[pl.ds(0, CH)], idst[p], semi[p]).wait()
      def s2_fire(kc, p):
        pltpu.async_copy(nf_hbm.at[isrc[p]], rows[p], semg[p])
      def s2_wait(p):
        pltpu.make_async_copy(nf_hbm.at[isrc[p]], rows[p], semg[p]).wait()
      def work_fire(kc, p):
        base = estart + kc * CH
        pltpu.async_copy(rows[p], g0_hbm.at[pl.ds(base, CH)], semw[p])
        pltpu.async_copy(rows[p], acc.at[idst[p]], sema[p], add=True)
      def work_wait(p):
        pltpu.make_async_copy(rows[p], g0_hbm.at[pl.ds(0, CH)], semw[p]).wait()
        pltpu.make_async_copy(rows[p], acc.at[idst[p]], sema[p]).wait()
      _pipeline(NCH, idx_fire, idx_wait, s2_fire, s2_wait, work_fire, work_wait)
      plsc.subcore_barrier()
      _writeout_shared(acc, s1_hbm, sid)

    @pl.when(cid == 1)
    def _():
      def idx_fire(kc, p):
        base = estart + kc * CH
        pltpu.async_copy(dst_hbm.at[pl.ds(base, CH)], idst[p], semi[p])
      def idx_wait(p):
        pltpu.make_async_copy(dst_hbm.at[pl.ds(0, CH)], idst[p], semi[p]).wait()
      def s2_fire(kc, p):
        base = estart + kc * CH
        pltpu.async_copy(nf_hbm.at[idst[p]], rows[p], semg[p])
        pltpu.async_copy(ef_hbm.at[pl.ds(base, CH)], erows[p], seme[p])
      def s2_wait(p):
        pltpu.make_async_copy(nf_hbm.at[idst[p]], rows[p], semg[p]).wait()
        pltpu.make_async_copy(ef_hbm.at[pl.ds(0, CH)], erows[p], seme[p]).wait()
      def work_fire(kc, p):
        base = estart + kc * CH
        pltpu.async_copy(rows[p], g1_hbm.at[pl.ds(base, CH)], semw[p])
        pltpu.async_copy(erows[p], acc.at[idst[p]], sema[p], add=True)
      def work_wait(p):
        pltpu.make_async_copy(rows[p], g1_hbm.at[pl.ds(0, CH)], semw[p]).wait()
        pltpu.make_async_copy(erows[p], acc.at[idst[p]], sema[p]).wait()
      _pipeline(NCH, idx_fire, idx_wait, s2_fire, s2_wait, work_fire, work_wait)
      plsc.subcore_barrier()
      _writeout_shared(acc, s2_hbm, sid)

  return k(nf, ef, src, dst)


def _prep_body(h_ref, nf_ref, w1aT_ref, b1_ref, w2aT_ref, b2_ref,
               re1_ref, re2_ref):
  f32 = jnp.float32
  hs = h_ref[0] + h_ref[1]
  maskT = (hs[:, 0:NUM_RELS] > 0).astype(f32)
  counts = jnp.sum(maskT, axis=0)
  nf = nf_ref[...]
  sums = []
  for r in range(NUM_RELS):
    sums.append(jnp.sum(nf * maskT[:, r:r + 1], axis=0, keepdims=True))
  sums = jnp.concatenate(sums, axis=0)
  rel = jnp.where(counts[:, None] > 0,
                  sums / jnp.maximum(counts, 1.0)[:, None], 0.0)
  re1_ref[...] = jnp.dot(rel, w1aT_ref[...],
                         preferred_element_type=f32) + b1_ref[...]
  re2_ref[...] = jnp.dot(rel, w2aT_ref[...],
                         preferred_element_type=f32) + b2_ref[...]


def _prep(h2, nf, w1aT, b1, w2aT, b2):
  return pl.pallas_call(
      _prep_body,
      out_shape=(jax.ShapeDtypeStruct((NUM_RELS, D), jnp.float32),
                 jax.ShapeDtypeStruct((NUM_RELS, D), jnp.float32)),
  )(h2, nf, w1aT, b1, w2aT, b2)


def _gru_combine(gi, gh, h):
  ir, iz, inn = gi[:, 0:D], gi[:, D:2 * D], gi[:, 2 * D:3 * D]
  hr, hz, hn = gh[:, 0:D], gh[:, D:2 * D], gh[:, 2 * D:3 * D]
  r = jax.nn.sigmoid(ir + hr)
  z = jax.nn.sigmoid(iz + hz)
  n = jnp.tanh(inn + r * hn)
  return (1.0 - z) * n + z * h


def _edge_body(et_ref, g0_ref, g1_ref, ef_ref, re2_ref, w2bT_ref, w2cT_ref,
               wihT_ref, whhT_ref, bih_ref, bhh_ref, out_ref):
  f32, bf16 = jnp.float32, jnp.bfloat16
  ef = ef_ref[...]
  efb = ef.astype(bf16)
  oh = (et_ref[...] == lax.broadcasted_iota(
      jnp.int32, (B_EDGE, NUM_RELS), 1)).astype(f32)
  x2 = jnp.dot(oh, re2_ref[...], preferred_element_type=f32)
  x2 = x2 + jnp.dot(g0_ref[...].astype(bf16), w2bT_ref[...].astype(bf16),
                    preferred_element_type=f32)
  x2 = x2 + jnp.dot(g1_ref[...].astype(bf16), w2cT_ref[...].astype(bf16),
                    preferred_element_type=f32)
  wihT = wihT_ref[...].astype(bf16)
  bih = bih_ref[...]
  gh = jnp.dot(efb, whhT_ref[...].astype(bf16),
               preferred_element_type=f32) + bhh_ref[...]
  gi = jnp.dot(x2.astype(bf16), wihT, preferred_element_type=f32) + bih
  e_msg = _gru_combine(gi, gh, ef)
  gi2 = jnp.dot(e_msg.astype(bf16), wihT, preferred_element_type=f32) + bih
  out_ref[...] = _gru_combine(gi2, gh, ef)


def _edge(et_col, g0, g1, ef, re2, w2bT, w2cT, wihT, whhT, bih, bhh):
  full = lambda shape: pl.BlockSpec(shape, lambda i: (0, 0))
  blk = lambda shape: pl.BlockSpec(shape, lambda i: (i, 0))
  return pl.pallas_call(
      _edge_body,
      grid=(NBLK,),
      in_specs=[
          blk((B_EDGE, 1)),
          blk((B_EDGE, D)),
          blk((B_EDGE, D)),
          blk((B_EDGE, D)),
          full((NUM_RELS, D)),
          full((D, D)),
          full((D, D)),
          full((D, 3 * D)),
          full((D, 3 * D)),
          full((1, 3 * D)),
          full((1, 3 * D)),
      ],
      out_specs=blk((B_EDGE, D)),
      out_shape=jax.ShapeDtypeStruct((N_EDGES, D), jnp.float32),
  )(et_col, g0, g1, ef, re2, w2bT, w2cT, wihT, whhT, bih, bhh)


def _node_body(nf_ref, s1_ref, s2_ref, h_ref, re1_ref, w1bT_ref, w1cT_ref,
               w3T_ref, b3_ref, out_ref):
  f32 = jnp.float32
  hs = h_ref[0] + h_ref[1]
  cnt2 = hs[:, NUM_RELS:2 * NUM_RELS]
  deg = jnp.sum(cnt2, axis=1, keepdims=True)
  aggs = jnp.dot(cnt2, re1_ref[...], preferred_element_type=f32)
  aggs = aggs + jnp.dot(s1_ref[...], w1bT_ref[...], preferred_element_type=f32)
  aggs = aggs + jnp.dot(s2_ref[...], w1cT_ref[...], preferred_element_type=f32)
  agg = aggs / jnp.maximum(deg, 1.0)
  x = agg + jnp.dot(nf_ref[...], w3T_ref[...],
                    preferred_element_type=f32) + b3_ref[...]
  out_ref[...] = jnp.where(x >= 0, x, RRELU_SLOPE * x)


def _node(nf, s1, s2, h2, re1, w1bT, w1cT, w3T, b3):
  return pl.pallas_call(
      _node_body,
      out_shape=jax.ShapeDtypeStruct((N_NODES, D), jnp.float32),
  )(nf, s1, s2, h2, re1, w1bT, w1cT, w3T, b3)


def kernel(node_feats, edge_index, edge_feats, edge_types, params):
  src = edge_index[0]
  dst = edge_index[1]
  et = edge_types
  et_col = et.reshape(N_EDGES, 1)

  eye8 = jnp.eye(NUM_RELS, dtype=jnp.float32)
  t1 = jnp.pad(eye8, ((0, 0), (0, D - NUM_RELS)))
  t2 = t1 + jnp.pad(eye8, ((0, 0), (NUM_RELS, D - 2 * NUM_RELS)))
  h2 = _hist_kernel(src, dst, et, t1, t2)

  nf, ef = node_feats, edge_feats
  for p in params:
    w1, w2 = p['W1'], p['W2']
    w1aT, w1bT, w1cT = w1[:, 0:D].T, w1[:, D:2 * D].T, w1[:, 2 * D:3 * D].T
    w2aT, w2bT, w2cT = w2[:, 0:D].T, w2[:, D:2 * D].T, w2[:, 2 * D:3 * D].T
    b1 = p['b1'].reshape(1, D)
    b2 = p['b2'].reshape(1, D)
    b3 = p['b3'].reshape(1, D)
    w3T = p['W3'].T
    wihT = p['Wih'].T
    whhT = p['Whh'].T
    bih = p['bih'].reshape(1, 3 * D)
    bhh = p['bhh'].reshape(1, 3 * D)

    g0, g1, s1, s2 = _gather_kernel(nf, ef, src, dst)
    re1, re2 = _prep(h2, nf, w1aT, b1, w2aT, b2)
    new_e = _edge(et_col, g0, g1, ef, re2, w2bT, w2cT, wihT, whhT, bih, bhh)
    new_n = _node(nf, s1, s2, h2, re1, w1bT, w1cT, w3T, b3)
    nf, ef = new_n, new_e
  return (nf, ef)


# speedup vs baseline: 4.5169x; 1.5622x over previous
import functools

import jax
import jax.numpy as jnp
from jax import lax
from jax.experimental import pallas as pl
from jax.experimental.pallas import tpu as pltpu
from jax.experimental.pallas import tpu_sc as plsc

N_NODES = 10000
N_EDGES = 160000
D = 128
NUM_RELS = 8
RRELU_SLOPE = (1.0 / 8.0 + 1.0 / 3.0) / 2.0

NC, NS = 2, 16
NW = NC * NS
CH = 80
EDGES_PER_SUB = N_EDGES // NS
NCH = EDGES_PER_SUB // CH
ROWS_PER_SUB = N_NODES // NS

B_EDGE = 2000
NBLK = N_EDGES // B_EDGE
_T_REP = 64

def _sc_mesh():
  return plsc.VectorSubcoreMesh(
      core_axis_name="c", subcore_axis_name="s", num_cores=NC, num_subcores=NS)


def _sc_params():
  import dataclasses
  cp = pltpu.CompilerParams()
  if "needs_layout_passes" in pltpu.CompilerParams.__dataclass_fields__:
    cp = dataclasses.replace(cp, needs_layout_passes=False)
  return cp


def _zero_rows(ref, nrows, ncols):
  @pl.loop(0, nrows)
  def _(i):
    for c in range(ncols // 16):
      ref[i, pl.ds(c * 16, 16)] = jnp.zeros((16,), jnp.float32)


def _pipeline(nch, idx_fire, idx_wait, s2_fire, s2_wait, work_fire, work_wait):
  assert nch >= 4 and nch % 2 == 1
  idx_fire(0, 0)
  idx_fire(1, 1)
  idx_wait(0)
  s2_fire(0, 0)

  def body(k, p, mid, tail):
    q = 1 - p
    s2_wait(p)
    work_fire(k, p)
    if mid:
      idx_wait(q)
      s2_fire(k + 1, q)
    work_wait(p)
    if tail:
      idx_fire(k + 2, p)

  @pl.loop(0, (nch - 3) // 2)
  def _(t):
    body(2 * t, 0, True, True)
    body(2 * t + 1, 1, True, True)

  body(nch - 3, 0, True, True)
  body(nch - 2, 1, True, False)
  body(nch - 1, 0, False, False)


_SUB_ROWS = 624
_TAIL_START = _SUB_ROWS * NS
_TAIL_ROWS = N_NODES - _TAIL_START


def _zero_shared_range(zbuf, shared, sid):
  start = pl.multiple_of(sid * _SUB_ROWS, 8)
  for i in range(7):
    pltpu.sync_copy(zbuf, shared.at[pl.ds(start + i * CH, CH)])
  pltpu.sync_copy(zbuf.at[pl.ds(0, 64)], shared.at[pl.ds(start + 560, 64)])
  @pl.when(sid == NS - 1)
  def _():
    pltpu.sync_copy(zbuf.at[pl.ds(0, _TAIL_ROWS)],
                    shared.at[pl.ds(_TAIL_START, _TAIL_ROWS)])


def _writeout_shared(shared, out_ref, sid):
  start = pl.multiple_of(sid * _SUB_ROWS, 8)
  pltpu.sync_copy(shared.at[pl.ds(start, _SUB_ROWS)],
                  out_ref.at[pl.ds(start, _SUB_ROWS)])
  @pl.when(sid == NS - 1)
  def _():
    pltpu.sync_copy(shared.at[pl.ds(_TAIL_START, _TAIL_ROWS)],
                    out_ref.at[pl.ds(_TAIL_START, _TAIL_ROWS)])


def _hist_kernel(src, dst, et, t1, t2):
  @functools.partial(
      pl.kernel,
      out_type=jax.ShapeDtypeStruct((NC, N_NODES, D), jnp.float32),
      mesh=_sc_mesh(),
      compiler_params=_sc_params(),
      scratch_types=[
          [pltpu.VMEM((CH,), jnp.int32)] * 2,
          [pltpu.VMEM((CH,), jnp.int32)] * 2,
          [pltpu.VMEM((CH, D), jnp.float32)] * 2,
          pltpu.VMEM_SHARED((N_NODES, D), jnp.float32),
          [pltpu.SemaphoreType.DMA] * 2,
          [pltpu.SemaphoreType.DMA] * 2,
          [pltpu.SemaphoreType.DMA] * 2,
      ],
  )
  def k(t1_hbm, t2_hbm, src_hbm, dst_hbm, et_hbm, h_hbm,
        iet, ind, rows, acc, semi, semg, semw):
    cid = lax.axis_index("c")
    sid = lax.axis_index("s")
    estart = sid * EDGES_PER_SUB

    _zero_rows(rows[0], CH, D)
    _zero_shared_range(rows[0], acc, sid)
    plsc.subcore_barrier()

    iota16 = lax.iota(jnp.int32, 16)

    def job(t_hbm, nd_hbm, out_ref):
      def idx_fire(kc, p):
        base = estart + kc * CH
        pltpu.async_copy(et_hbm.at[pl.ds(base, CH)], iet[p], semi[p])
        pltpu.async_copy(nd_hbm.at[pl.ds(base, CH)], ind[p], semi[p])
      def idx_wait(p):
        pltpu.make_async_copy(et_hbm.at[pl.ds(0, CH)], iet[p], semi[p]).wait()
        pltpu.make_async_copy(nd_hbm.at[pl.ds(0, CH)], ind[p], semi[p]).wait()
      def s2_fire(kc, p):
        @pl.loop(0, CH, step=16)
        def _(j):
          v = iet[p][pl.ds(j, 16)]
          rep = (iota16 + j + kc) & (_T_REP - 1)
          iet[p][pl.ds(j, 16)] = v + rep * NUM_RELS
        pltpu.async_copy(t_hbm.at[iet[p]], rows[p], semg[p])
      def s2_wait(p):
        pltpu.make_async_copy(t_hbm.at[iet[p]], rows[p], semg[p]).wait()
      def work_fire(kc, p):
        pltpu.async_copy(rows[p], acc.at[ind[p]], semw[p], add=True)
      def work_wait(p):
        pltpu.make_async_copy(rows[p], acc.at[ind[p]], semw[p]).wait()
      _pipeline(NCH, idx_fire, idx_wait, s2_fire, s2_wait, work_fire, work_wait)
      plsc.subcore_barrier()
      _writeout_shared(acc, out_ref, sid)

    @pl.when(cid == 0)
    def _():
      job(t1_hbm, src_hbm, h_hbm.at[0])

    @pl.when(cid == 1)
    def _():
      job(t2_hbm, dst_hbm, h_hbm.at[1])

  return k(t1, t2, src, dst, et)


def _gather_kernel(nf, ef, src, dst):
  @functools.partial(
      pl.kernel,
      out_type=(
          jax.ShapeDtypeStruct((N_EDGES, D), jnp.float32),
          jax.ShapeDtypeStruct((N_EDGES, D), jnp.float32),
          jax.ShapeDtypeStruct((N_NODES, D), jnp.float32),
          jax.ShapeDtypeStruct((N_NODES, D), jnp.float32),
      ),
      mesh=_sc_mesh(),
      compiler_params=_sc_params(),
      scratch_types=[
          [pltpu.VMEM((CH,), jnp.int32)] * 2,
          [pltpu.VMEM((CH,), jnp.int32)] * 2,
          [pltpu.VMEM((CH, D), jnp.float32)] * 2,
          [pltpu.VMEM((CH, D), jnp.float32)] * 2,
          pltpu.VMEM_SHARED((N_NODES, D), jnp.float32),
          [pltpu.SemaphoreType.DMA] * 2,
          [pltpu.SemaphoreType.DMA] * 2,
          [pltpu.SemaphoreType.DMA] * 2,
          [pltpu.SemaphoreType.DMA] * 2,
          [pltpu.SemaphoreType.DMA] * 2,
      ],
  )
  def k(nf_hbm, ef_hbm, src_hbm, dst_hbm, g0_hbm, g1_hbm, s1_hbm, s2_hbm,
        isrc, idst, rows, erows, acc, semi, semg, seme, semw, sema):
    cid = lax.axis_index("c")
    sid = lax.axis_index("s")
    estart = sid * EDGES_PER_SUB

    _zero_rows(rows[0], CH, D)
    _zero_shared_range(rows[0], acc, sid)
    plsc.subcore_barrier()

    @pl.when(cid == 0)
    def _():
      def idx_fire(kc, p):
        base = estart + kc * CH
        pltpu.async_copy(src_hbm.at[pl.ds(base, CH)], isrc[p], semi[p])
        pltpu.async_copy(dst_hbm.at[pl.ds(base, CH)], idst[p], semi[p])
      def idx_wait(p):
        pltpu.make_async_copy(src_hbm.at[pl.ds(0, CH)], isrc[p], semi[p]).wait()
        pltpu.make_async_copy(dst_hbm.at[pl.ds(0, CH)], idst[p], semi[p]).wait()
      def s2_fire(kc, p):
        pltpu.async_copy(nf_hbm.at[isrc[p]], rows[p], semg[p])
      def s2_wait(p):
        pltpu.make_async_copy(nf_hbm.at[isrc[p]], rows[p], semg[p]).wait()
      def work_fire(kc, p):
        base = estart + kc * CH
        pltpu.async_copy(rows[p], g0_hbm.at[pl.ds(base, CH)], semw[p])
        pltpu.async_copy(rows[p], acc.at[idst[p]], sema[p], add=True)
      def work_wait(p):
        pltpu.make_async_copy(rows[p], g0_hbm.at[pl.ds(0, CH)], semw[p]).wait()
        pltpu.make_async_copy(rows[p], acc.at[idst[p]], sema[p]).wait()
      _pipeline(NCH, idx_fire, idx_wait, s2_fire, s2_wait, work_fire, work_wait)
      plsc.subcore_barrier()
      _writeout_shared(acc, s1_hbm, sid)

    @pl.when(cid == 1)
    def _():
      def idx_fire(kc, p):
        base = estart + kc * CH
        pltpu.async_copy(dst_hbm.at[pl.ds(base, CH)], idst[p], semi[p])
      def idx_wait(p):
        pltpu.make_async_copy(dst_hbm.at[pl.ds(0, CH)], idst[p], semi[p]).wait()
      def s2_fire(kc, p):
        base = estart + kc * CH
        pltpu.async_copy(nf_hbm.at[idst[p]], rows[p], semg[p])
        pltpu.async_copy(ef_hbm.at[pl.ds(base, CH)], erows[p], seme[p])
      def s2_wait(p):
        pltpu.make_async_copy(nf_hbm.at[idst[p]], rows[p], semg[p]).wait()
        pltpu.make_async_copy(ef_hbm.at[pl.ds(0, CH)], erows[p], seme[p]).wait()
      def work_fire(kc, p):
        base = estart + kc * CH
        pltpu.async_copy(rows[p], g1_hbm.at[pl.ds(base, CH)], semw[p])
        pltpu.async_copy(erows[p], acc.at[idst[p]], sema[p], add=True)
      def work_wait(p):
        pltpu.make_async_copy(rows[p], g1_hbm.at[pl.ds(0, CH)], semw[p]).wait()
        pltpu.make_async_copy(erows[p], acc.at[idst[p]], sema[p]).wait()
      _pipeline(NCH, idx_fire, idx_wait, s2_fire, s2_wait, work_fire, work_wait)
      plsc.subcore_barrier()
      _writeout_shared(acc, s2_hbm, sid)

  return k(nf, ef, src, dst)


def _prep_body(h_ref, nf_ref, w1aT_ref, b1_ref, w2aT_ref, b2_ref,
               re1_ref, re2_ref):
  f32 = jnp.float32
  hs = h_ref[0] + h_ref[1]
  maskT = (hs[:, 0:NUM_RELS] > 0).astype(f32)
  counts = jnp.sum(maskT, axis=0)
  nf = nf_ref[...]
  sums = []
  for r in range(NUM_RELS):
    sums.append(jnp.sum(nf * maskT[:, r:r + 1], axis=0, keepdims=True))
  sums = jnp.concatenate(sums, axis=0)
  rel = jnp.where(counts[:, None] > 0,
                  sums / jnp.maximum(counts, 1.0)[:, None], 0.0)
  re1_ref[...] = jnp.dot(rel, w1aT_ref[...],
                         preferred_element_type=f32) + b1_ref[...]
  re2_ref[...] = jnp.dot(rel, w2aT_ref[...],
                         preferred_element_type=f32) + b2_ref[...]


def _prep(h2, nf, w1aT, b1, w2aT, b2):
  return pl.pallas_call(
      _prep_body,
      out_shape=(jax.ShapeDtypeStruct((NUM_RELS, D), jnp.float32),
                 jax.ShapeDtypeStruct((NUM_RELS, D), jnp.float32)),
  )(h2, nf, w1aT, b1, w2aT, b2)


def _gru_combine(gi, gh, h):
  ir, iz, inn = gi[:, 0:D], gi[:, D:2 * D], gi[:, 2 * D:3 * D]
  hr, hz, hn = gh[:, 0:D], gh[:, D:2 * D], gh[:, 2 * D:3 * D]
  r = jax.nn.sigmoid(ir + hr)
  z = jax.nn.sigmoid(iz + hz)
  n = jnp.tanh(inn + r * hn)
  return (1.0 - z) * n + z * h


def _edge_body(et_ref, g0_ref, g1_ref, ef_ref, re2_ref, w2bT_ref, w2cT_ref,
               wihT_ref, whhT_ref, bih_ref, bhh_ref, out_ref):
  f32, bf16 = jnp.float32, jnp.bfloat16
  ef = ef_ref[...]
  efb = ef.astype(bf16)
  oh = (et_ref[...] == lax.broadcasted_iota(
      jnp.int32, (B_EDGE, NUM_RELS), 1)).astype(f32)
  x2 = jnp.dot(oh, re2_ref[...], preferred_element_type=f32)
  x2 = x2 + jnp.dot(g0_ref[...].astype(bf16), w2bT_ref[...].astype(bf16),
                    preferred_element_type=f32)
  x2 = x2 + jnp.dot(g1_ref[...].astype(bf16), w2cT_ref[...].astype(bf16),
                    preferred_element_type=f32)
  wihT = wihT_ref[...].astype(bf16)
  bih = bih_ref[...]
  gh = jnp.dot(efb, whhT_ref[...].astype(bf16),
               preferred_element_type=f32) + bhh_ref[...]
  gi = jnp.dot(x2.astype(bf16), wihT, preferred_element_type=f32) + bih
  e_msg = _gru_combine(gi, gh, ef)
  gi2 = jnp.dot(e_msg.astype(bf16), wihT, preferred_element_type=f32) + bih
  out_ref[...] = _gru_combine(gi2, gh, ef)


def _edge(et_col, g0, g1, ef, re2, w2bT, w2cT, wihT, whhT, bih, bhh):
  full = lambda shape: pl.BlockSpec(shape, lambda i: (0, 0))
  blk = lambda shape: pl.BlockSpec(shape, lambda i: (i, 0))
  return pl.pallas_call(
      _edge_body,
      grid=(NBLK,),
      in_specs=[
          blk((B_EDGE, 1)),
          blk((B_EDGE, D)),
          blk((B_EDGE, D)),
          blk((B_EDGE, D)),
          full((NUM_RELS, D)),
          full((D, D)),
          full((D, D)),
          full((D, 3 * D)),
          full((D, 3 * D)),
          full((1, 3 * D)),
          full((1, 3 * D)),
      ],
      out_specs=blk((B_EDGE, D)),
      out_shape=jax.ShapeDtypeStruct((N_EDGES, D), jnp.float32),
  )(et_col, g0, g1, ef, re2, w2bT, w2cT, wihT, whhT, bih, bhh)


def _node_body(nf_ref, s1_ref, s2_ref, h_ref, re1_ref, w1bT_ref, w1cT_ref,
               w3T_ref, b3_ref, out_ref):
  f32 = jnp.float32
  hs = h_ref[0] + h_ref[1]
  cnt2 = hs[:, NUM_RELS:2 * NUM_RELS]
  deg = jnp.sum(cnt2, axis=1, keepdims=True)
  aggs = jnp.dot(cnt2, re1_ref[...], preferred_element_type=f32)
  aggs = aggs + jnp.dot(s1_ref[...], w1bT_ref[...], preferred_element_type=f32)
  aggs = aggs + jnp.dot(s2_ref[...], w1cT_ref[...], preferred_element_type=f32)
  agg = aggs / jnp.maximum(deg, 1.0)
  x = agg + jnp.dot(nf_ref[...], w3T_ref[...],
                    preferred_element_type=f32) + b3_ref[...]
  out_ref[...] = jnp.where(x >= 0, x, RRELU_SLOPE * x)


def _node(nf, s1, s2, h2, re1, w1bT, w1cT, w3T, b3):
  return pl.pallas_call(
      _node_body,
      out_shape=jax.ShapeDtypeStruct((N_NODES, D), jnp.float32),
  )(nf, s1, s2, h2, re1, w1bT, w1cT, w3T, b3)


def kernel(node_feats, edge_index, edge_feats, edge_types, params):
  src = edge_index[0]
  dst = edge_index[1]
  et = edge_types
  et_col = et.reshape(N_EDGES, 1)

  eye8 = jnp.eye(NUM_RELS, dtype=jnp.float32)
  t1 = jnp.pad(eye8, ((0, 0), (0, D - NUM_RELS)))
  t2 = t1 + jnp.pad(eye8, ((0, 0), (NUM_RELS, D - 2 * NUM_RELS)))
  t1 = jnp.tile(t1, (_T_REP, 1))
  t2 = jnp.tile(t2, (_T_REP, 1))
  h2 = _hist_kernel(src, dst, et, t1, t2)

  nf, ef = node_feats, edge_feats
  for p in params:
    w1, w2 = p['W1'], p['W2']
    w1aT, w1bT, w1cT = w1[:, 0:D].T, w1[:, D:2 * D].T, w1[:, 2 * D:3 * D].T
    w2aT, w2bT, w2cT = w2[:, 0:D].T, w2[:, D:2 * D].T, w2[:, 2 * D:3 * D].T
    b1 = p['b1'].reshape(1, D)
    b2 = p['b2'].reshape(1, D)
    b3 = p['b3'].reshape(1, D)
    w3T = p['W3'].T
    wihT = p['Wih'].T
    whhT = p['Whh'].T
    bih = p['bih'].reshape(1, 3 * D)
    bhh = p['bhh'].reshape(1, 3 * D)

    g0, g1, s1, s2 = _gather_kernel(nf, ef, src, dst)
    re1, re2 = _prep(h2, nf, w1aT, b1, w2aT, b2)
    new_e = _edge(et_col, g0, g1, ef, re2, w2bT, w2cT, wihT, whhT, bih, bhh)
    new_n = _node(nf, s1, s2, h2, re1, w1bT, w1cT, w3T, b3)
    nf, ef = new_n, new_e
  return (nf, ef)
